# baseline (speedup 1.0000x reference)
import jax
import jax.numpy as jnp
from jax.experimental import pallas as pl
from jax.experimental.pallas import tpu as pltpu


def kernel(tokens, W_pos):
    batch, seq = tokens.shape
    d = W_pos.shape[1]
    blk = 1024
    nj = seq // blk

    def body(w0_ref, w1_ref, o_ref, sem):
        j = pl.program_id(0)

        def mk(b, off):
            src = w0_ref if b < batch // 2 else w1_ref
            return pltpu.make_async_copy(
                src, o_ref.at[b, pl.ds(off * blk, blk), :], sem.at[b]
            )

        @pl.when(j >= 1)
        def _():
            for b in range(batch):
                mk(b, j - 1).wait()

        for b in range(batch):
            mk(b, j).start()

        @pl.when(j == nj - 1)
        def _():
            for b in range(batch):
                mk(b, j).wait()

    out = pl.pallas_call(
        body,
        grid=(nj,),
        in_specs=[
            pl.BlockSpec((blk, d), lambda j: (j, 0)),
            pl.BlockSpec((blk, d), lambda j: (j, 0)),
        ],
        out_specs=pl.BlockSpec(memory_space=pl.ANY),
        out_shape=jax.ShapeDtypeStruct((batch, seq, d), W_pos.dtype),
        scratch_shapes=[pltpu.SemaphoreType.DMA((batch,))],
    )(W_pos, W_pos)
    return out


# speedup vs baseline: 1.2471x; 1.2471x over previous
import jax
import jax.numpy as jnp
from jax.experimental import pallas as pl


def _copy_body(w_ref, o_ref):
    o_ref[...] = jnp.broadcast_to(w_ref[...][None], o_ref.shape)


def kernel(tokens, W_pos):
    batch, seq = tokens.shape
    d = W_pos.shape[1]
    blk = 1024
    out = pl.pallas_call(
        _copy_body,
        grid=(seq // blk,),
        in_specs=[pl.BlockSpec((blk, d), lambda j: (j, 0))],
        out_specs=pl.BlockSpec((batch, blk, d), lambda j: (0, j, 0)),
        out_shape=jax.ShapeDtypeStruct((batch, seq, d), W_pos.dtype),
    )(W_pos)
    return out
